# baseline (speedup 1.0000x reference)
import functools

import jax
import jax.numpy as jnp
from jax import lax
from jax.experimental import pallas as pl
from jax.experimental.pallas import tpu as pltpu
from jax.experimental.pallas import tpu_sc as plsc

N_NODES = 10000
N_EDGES = 320000
DIM = 128
ALP = 1.0
LAM = 1.0
C_SELF = 1.0 - ALP - ALP * LAM
C_AGG = ALP * LAM
C_INI = ALP

NPAD = 10240
NC = 2
NS = 16
NW = NC * NS
ROWS_T = NPAD // NS

W_E = 84
EPT = 10080
NWIN = EPT // W_E
NBUF = 4
E_PAD = NW * EPT

W_H = 80
EPS = N_EDGES // NS
NWIN_H = EPS // W_H
RING_H = 10

TCB = 1000


def _degrees(e3, zeros1):
    mesh = plsc.VectorSubcoreMesh(core_axis_name="c", subcore_axis_name="s")

    @functools.partial(
        pl.kernel,
        out_type=jax.ShapeDtypeStruct((NC, NPAD), jnp.float32),
        mesh=mesh,
        scratch_types=(
            [pltpu.VMEM((NWIN_H, W_H), jnp.int32),
             pltpu.VMEM((W_H,), jnp.float32),
             pltpu.VMEM_SHARED((NPAD,), jnp.float32)]
            + [pltpu.SemaphoreType.DMA for _ in range(RING_H)]
        ),
    )
    def k(e_hbm, z_hbm, out_hbm, idx_v, ones_v, acc_sh, *hsem):
        c = lax.axis_index("c")
        s = lax.axis_index("s")
        seg = NPAD // NS

        @pl.loop(0, W_H, step=16)
        def _(i):
            ones_v[pl.ds(i, 16)] = jnp.ones((16,), jnp.float32)

        pltpu.sync_copy(z_hbm, acc_sh.at[pl.ds(s * seg, seg)])
        pltpu.sync_copy(e_hbm.at[c * NS + s], idx_v)
        plsc.subcore_barrier()

        @pl.loop(0, NWIN_H, step=RING_H)
        def _(w):
            cps = [pltpu.async_copy(ones_v, acc_sh.at[idx_v.at[w + b]],
                                    hsem[b], add=True)
                   for b in range(RING_H)]
            for cp in cps:
                cp.wait()

        plsc.subcore_barrier()
        pltpu.sync_copy(acc_sh.at[pl.ds(s * seg, seg)],
                        out_hbm.at[c, pl.ds(s * seg, seg)])

    return k(e3, zeros1)


def _propagate(table, src3, dst3, zeros2):
    mesh = plsc.VectorSubcoreMesh(core_axis_name="c", subcore_axis_name="s")

    @functools.partial(
        pl.kernel,
        out_type=jax.ShapeDtypeStruct((NC, NPAD, DIM), jnp.float32),
        mesh=mesh,
        scratch_types=(
            [pltpu.VMEM((W_E,), jnp.int32) for _ in range(2 * NBUF)]
            + [pltpu.VMEM((W_E, DIM), jnp.float32) for _ in range(NBUF)]
            + [pltpu.VMEM_SHARED((NPAD, DIM), jnp.float32)]
            + [pltpu.SemaphoreType.DMA for _ in range(4 * NBUF)]
        ),
    )
    def k(t_hbm, s_hbm, d_hbm, z_hbm, out_hbm, *rest):
        src_v = rest[:NBUF]
        dst_v = rest[NBUF:2 * NBUF]
        rows = rest[2 * NBUF:3 * NBUF]
        acc_sh = rest[3 * NBUF]
        sems = rest[3 * NBUF + 1:]
        isem = sems[:NBUF]
        jsem = sems[NBUF:2 * NBUF]
        gsem = sems[2 * NBUF:3 * NBUF]
        ssem = sems[3 * NBUF:]
        c = lax.axis_index("c")
        s = lax.axis_index("s")
        wid = c * NS + s

        def idx_load(w, b):
            return (pltpu.make_async_copy(s_hbm.at[wid, w], src_v[b], isem[b]),
                    pltpu.make_async_copy(d_hbm.at[wid, w], dst_v[b], jsem[b]))

        def gath(w, b):
            return pltpu.make_async_copy(t_hbm.at[src_v[b]], rows[b], gsem[b])

        def scat_start(b):
            return pltpu.async_copy(rows[b], acc_sh.at[dst_v[b]],
                                    ssem[b], add=True)

        for b in range(NBUF):
            a, d = idx_load(b, b)
            a.start()
            d.start()
        pltpu.sync_copy(z_hbm, acc_sh.at[pl.ds(s * ROWS_T, ROWS_T)])
        plsc.subcore_barrier()

        @pl.loop(0, NWIN, step=NBUF)
        def _(w):
            for b in range(NBUF):
                a, _d = idx_load(w + b, b)
                a.wait()
                gath(w + b, b).start()
            scats = []
            for b in range(NBUF):
                gath(w + b, b).wait()
                _a, d = idx_load(w + b, b)
                d.wait()
                scats.append(scat_start(b))
            for b in range(NBUF):
                scats[b].wait()

                @pl.when(w + b + NBUF < NWIN)
                def _():
                    a, d = idx_load(w + b + NBUF, b)
                    a.start()
                    d.start()

        plsc.subcore_barrier()
        pltpu.sync_copy(acc_sh.at[pl.ds(s * ROWS_T, ROWS_T)],
                        out_hbm.at[c, pl.ds(s * ROWS_T, ROWS_T)])

    return k(table, src3, dst3, zeros2)


def _mlp_scale(xp, w, b2, deg):
    def body(x_ref, w_ref, b_ref, d_ref, h_ref, t_ref):
        h = lax.dot_general(
            x_ref[...], w_ref[...], (((1,), (1,)), ((), ())),
            preferred_element_type=jnp.float32) + b_ref[...]
        n0 = lax.rsqrt(jnp.maximum(d_ref[:, 0], 1.0))
        h_ref[...] = h
        t_ref[...] = h * n0[:, None]

    return pl.pallas_call(
        body,
        out_shape=(jax.ShapeDtypeStruct((N_NODES, DIM), jnp.float32),
                   jax.ShapeDtypeStruct((N_NODES, DIM), jnp.float32)),
        grid=(N_NODES // TCB,),
        in_specs=[pl.BlockSpec((TCB, DIM), lambda i: (i, 0)),
                  pl.BlockSpec((DIM, DIM), lambda i: (0, 0)),
                  pl.BlockSpec((1, DIM), lambda i: (0, 0)),
                  pl.BlockSpec((TCB, NC), lambda i: (i, 0))],
        out_specs=(pl.BlockSpec((TCB, DIM), lambda i: (i, 0)),
                   pl.BlockSpec((TCB, DIM), lambda i: (i, 0))),
    )(xp, w, b2, deg)


def _mid(p, deg):
    def body(p0_ref, p1_ref, d_ref, t_ref):
        n0 = lax.rsqrt(jnp.maximum(d_ref[:, 0], 1.0))
        n1 = lax.rsqrt(jnp.maximum(d_ref[:, 1], 1.0))
        t_ref[...] = (p0_ref[0] + p1_ref[0]) * (n0 * n1)[:, None]

    return pl.pallas_call(
        body,
        out_shape=jax.ShapeDtypeStruct((N_NODES, DIM), jnp.float32),
        grid=(N_NODES // TCB,),
        in_specs=[pl.BlockSpec((1, TCB, DIM), lambda i: (0, i, 0)),
                  pl.BlockSpec((1, TCB, DIM), lambda i: (1, i, 0)),
                  pl.BlockSpec((TCB, NC), lambda i: (i, 0))],
        out_specs=pl.BlockSpec((TCB, DIM), lambda i: (i, 0)),
    )(p, p, deg)


def _last(q, p, h0, deg):
    def body(q0_ref, q1_ref, p0_ref, p1_ref, h_ref, d_ref, o_ref):
        n1 = lax.rsqrt(jnp.maximum(d_ref[:, 1], 1.0))
        diff = q0_ref[0] + q1_ref[0] - p0_ref[0] - p1_ref[0]
        o_ref[...] = diff * n1[:, None] + h_ref[...]

    return pl.pallas_call(
        body,
        out_shape=jax.ShapeDtypeStruct((N_NODES, DIM), jnp.float32),
        grid=(N_NODES // TCB,),
        in_specs=[pl.BlockSpec((1, TCB, DIM), lambda i: (0, i, 0)),
                  pl.BlockSpec((1, TCB, DIM), lambda i: (1, i, 0)),
                  pl.BlockSpec((1, TCB, DIM), lambda i: (0, i, 0)),
                  pl.BlockSpec((1, TCB, DIM), lambda i: (1, i, 0)),
                  pl.BlockSpec((TCB, DIM), lambda i: (i, 0)),
                  pl.BlockSpec((TCB, NC), lambda i: (i, 0))],
        out_specs=pl.BlockSpec((TCB, DIM), lambda i: (i, 0)),
    )(q, q, p, p, h0, deg)


def kernel(features, edge_index, W, b):
    e3 = edge_index.reshape(NW, NWIN_H, W_H)
    pad_n = E_PAD - N_EDGES
    pad_i = jnp.arange(pad_n, dtype=jnp.int32)
    src_pad = pad_i % jnp.int32(N_NODES)
    dst_pad = jnp.int32(N_NODES) + pad_i % jnp.int32(NPAD - N_NODES)
    src3 = jnp.concatenate([edge_index[0], src_pad]).reshape(NW, NWIN, W_E)
    dst3 = jnp.concatenate([edge_index[1], dst_pad]).reshape(NW, NWIN, W_E)
    zeros1 = jnp.zeros((ROWS_T,), jnp.float32)
    zeros2 = jnp.zeros((ROWS_T, DIM), jnp.float32)
    b2 = b.reshape(1, DIM)

    deg = _degrees(e3, zeros1).T
    h0, t1 = _mlp_scale(features, W, b2, deg)
    p = _propagate(t1, src3, dst3, zeros2)
    t2 = _mid(p, deg)
    q = _propagate(t2, src3, dst3, zeros2)
    return _last(q, p, h0, deg)


# speedup vs baseline: 1.0048x; 1.0048x over previous
import functools

import jax
import jax.numpy as jnp
from jax import lax
from jax.experimental import pallas as pl
from jax.experimental.pallas import tpu as pltpu
from jax.experimental.pallas import tpu_sc as plsc

N_NODES = 10000
N_EDGES = 320000
DIM = 128
ALP = 1.0
LAM = 1.0
C_SELF = 1.0 - ALP - ALP * LAM
C_AGG = ALP * LAM
C_INI = ALP

NPAD = 10240
NC = 2
NS = 16
NW = NC * NS
ROWS_T = NPAD // NS

W_E = 72
EPT = 10080
NWIN = EPT // W_E
NBUF = 5
E_PAD = NW * EPT

W_H = 80
EPS = N_EDGES // NS
NWIN_H = EPS // W_H
RING_H = 10

TCB = 1000


def _degrees(e3, zeros1):
    mesh = plsc.VectorSubcoreMesh(core_axis_name="c", subcore_axis_name="s")

    @functools.partial(
        pl.kernel,
        out_type=jax.ShapeDtypeStruct((NC, NPAD), jnp.float32),
        mesh=mesh,
        scratch_types=(
            [pltpu.VMEM((NWIN_H, W_H), jnp.int32),
             pltpu.VMEM((W_H,), jnp.float32),
             pltpu.VMEM_SHARED((NPAD,), jnp.float32)]
            + [pltpu.SemaphoreType.DMA for _ in range(RING_H)]
        ),
    )
    def k(e_hbm, z_hbm, out_hbm, idx_v, ones_v, acc_sh, *hsem):
        c = lax.axis_index("c")
        s = lax.axis_index("s")
        seg = NPAD // NS

        @pl.loop(0, W_H, step=16)
        def _(i):
            ones_v[pl.ds(i, 16)] = jnp.ones((16,), jnp.float32)

        pltpu.sync_copy(z_hbm, acc_sh.at[pl.ds(s * seg, seg)])
        pltpu.sync_copy(e_hbm.at[c * NS + s], idx_v)
        plsc.subcore_barrier()

        @pl.loop(0, NWIN_H, step=RING_H)
        def _(w):
            cps = [pltpu.async_copy(ones_v, acc_sh.at[idx_v.at[w + b]],
                                    hsem[b], add=True)
                   for b in range(RING_H)]
            for cp in cps:
                cp.wait()

        plsc.subcore_barrier()
        pltpu.sync_copy(acc_sh.at[pl.ds(s * seg, seg)],
                        out_hbm.at[c, pl.ds(s * seg, seg)])

    return k(e3, zeros1)


def _propagate(table, src3, dst3, zeros2):
    mesh = plsc.VectorSubcoreMesh(core_axis_name="c", subcore_axis_name="s")

    @functools.partial(
        pl.kernel,
        out_type=jax.ShapeDtypeStruct((NC, NPAD, DIM), jnp.float32),
        mesh=mesh,
        scratch_types=(
            [pltpu.VMEM((W_E,), jnp.int32) for _ in range(2 * NBUF)]
            + [pltpu.VMEM((W_E, DIM), jnp.float32) for _ in range(NBUF)]
            + [pltpu.VMEM_SHARED((NPAD, DIM), jnp.float32)]
            + [pltpu.SemaphoreType.DMA for _ in range(4 * NBUF)]
        ),
    )
    def k(t_hbm, s_hbm, d_hbm, z_hbm, out_hbm, *rest):
        src_v = rest[:NBUF]
        dst_v = rest[NBUF:2 * NBUF]
        rows = rest[2 * NBUF:3 * NBUF]
        acc_sh = rest[3 * NBUF]
        sems = rest[3 * NBUF + 1:]
        isem = sems[:NBUF]
        jsem = sems[NBUF:2 * NBUF]
        gsem = sems[2 * NBUF:3 * NBUF]
        ssem = sems[3 * NBUF:]
        c = lax.axis_index("c")
        s = lax.axis_index("s")
        wid = c * NS + s

        def idx_load(w, b):
            return (pltpu.make_async_copy(s_hbm.at[wid, w], src_v[b], isem[b]),
                    pltpu.make_async_copy(d_hbm.at[wid, w], dst_v[b], jsem[b]))

        def gath(w, b):
            return pltpu.make_async_copy(t_hbm.at[src_v[b]], rows[b], gsem[b])

        def scat_start(b):
            return pltpu.async_copy(rows[b], acc_sh.at[dst_v[b]],
                                    ssem[b], add=True)

        for b in range(NBUF):
            a, d = idx_load(b, b)
            a.start()
            d.start()
        pltpu.sync_copy(z_hbm, acc_sh.at[pl.ds(s * ROWS_T, ROWS_T)])
        plsc.subcore_barrier()

        @pl.loop(0, NWIN, step=NBUF)
        def _(w):
            for b in range(NBUF):
                a, _d = idx_load(w + b, b)
                a.wait()
                gath(w + b, b).start()
            scats = []
            for b in range(NBUF):
                gath(w + b, b).wait()
                _a, d = idx_load(w + b, b)
                d.wait()
                scats.append(scat_start(b))
            for b in range(NBUF):
                scats[b].wait()

                @pl.when(w + b + NBUF < NWIN)
                def _():
                    a, d = idx_load(w + b + NBUF, b)
                    a.start()
                    d.start()

        plsc.subcore_barrier()
        pltpu.sync_copy(acc_sh.at[pl.ds(s * ROWS_T, ROWS_T)],
                        out_hbm.at[c, pl.ds(s * ROWS_T, ROWS_T)])

    return k(table, src3, dst3, zeros2)


def _mlp_scale(xp, w, b2, deg):
    def body(x_ref, w_ref, b_ref, d_ref, h_ref, t_ref):
        h = lax.dot_general(
            x_ref[...], w_ref[...], (((1,), (1,)), ((), ())),
            preferred_element_type=jnp.float32) + b_ref[...]
        n0 = lax.rsqrt(jnp.maximum(d_ref[:, 0], 1.0))
        h_ref[...] = h
        t_ref[...] = h * n0[:, None]

    return pl.pallas_call(
        body,
        out_shape=(jax.ShapeDtypeStruct((N_NODES, DIM), jnp.float32),
                   jax.ShapeDtypeStruct((N_NODES, DIM), jnp.float32)),
        grid=(N_NODES // TCB,),
        in_specs=[pl.BlockSpec((TCB, DIM), lambda i: (i, 0)),
                  pl.BlockSpec((DIM, DIM), lambda i: (0, 0)),
                  pl.BlockSpec((1, DIM), lambda i: (0, 0)),
                  pl.BlockSpec((TCB, NC), lambda i: (i, 0))],
        out_specs=(pl.BlockSpec((TCB, DIM), lambda i: (i, 0)),
                   pl.BlockSpec((TCB, DIM), lambda i: (i, 0))),
    )(xp, w, b2, deg)


def _mid(p, deg):
    def body(p0_ref, p1_ref, d_ref, t_ref):
        n0 = lax.rsqrt(jnp.maximum(d_ref[:, 0], 1.0))
        n1 = lax.rsqrt(jnp.maximum(d_ref[:, 1], 1.0))
        t_ref[...] = (p0_ref[0] + p1_ref[0]) * (n0 * n1)[:, None]

    return pl.pallas_call(
        body,
        out_shape=jax.ShapeDtypeStruct((N_NODES, DIM), jnp.float32),
        grid=(N_NODES // TCB,),
        in_specs=[pl.BlockSpec((1, TCB, DIM), lambda i: (0, i, 0)),
                  pl.BlockSpec((1, TCB, DIM), lambda i: (1, i, 0)),
                  pl.BlockSpec((TCB, NC), lambda i: (i, 0))],
        out_specs=pl.BlockSpec((TCB, DIM), lambda i: (i, 0)),
    )(p, p, deg)


def _last(q, p, h0, deg):
    def body(q0_ref, q1_ref, p0_ref, p1_ref, h_ref, d_ref, o_ref):
        n1 = lax.rsqrt(jnp.maximum(d_ref[:, 1], 1.0))
        diff = q0_ref[0] + q1_ref[0] - p0_ref[0] - p1_ref[0]
        o_ref[...] = diff * n1[:, None] + h_ref[...]

    return pl.pallas_call(
        body,
        out_shape=jax.ShapeDtypeStruct((N_NODES, DIM), jnp.float32),
        grid=(N_NODES // TCB,),
        in_specs=[pl.BlockSpec((1, TCB, DIM), lambda i: (0, i, 0)),
                  pl.BlockSpec((1, TCB, DIM), lambda i: (1, i, 0)),
                  pl.BlockSpec((1, TCB, DIM), lambda i: (0, i, 0)),
                  pl.BlockSpec((1, TCB, DIM), lambda i: (1, i, 0)),
                  pl.BlockSpec((TCB, DIM), lambda i: (i, 0)),
                  pl.BlockSpec((TCB, NC), lambda i: (i, 0))],
        out_specs=pl.BlockSpec((TCB, DIM), lambda i: (i, 0)),
    )(q, q, p, p, h0, deg)


def kernel(features, edge_index, W, b):
    e3 = edge_index.reshape(NW, NWIN_H, W_H)
    pad_n = E_PAD - N_EDGES
    pad_i = jnp.arange(pad_n, dtype=jnp.int32)
    src_pad = pad_i % jnp.int32(N_NODES)
    dst_pad = jnp.int32(N_NODES) + pad_i % jnp.int32(NPAD - N_NODES)
    src3 = jnp.concatenate([edge_index[0], src_pad]).reshape(NW, NWIN, W_E)
    dst3 = jnp.concatenate([edge_index[1], dst_pad]).reshape(NW, NWIN, W_E)
    zeros1 = jnp.zeros((ROWS_T,), jnp.float32)
    zeros2 = jnp.zeros((ROWS_T, DIM), jnp.float32)
    b2 = b.reshape(1, DIM)

    deg = _degrees(e3, zeros1).T
    h0, t1 = _mlp_scale(features, W, b2, deg)
    p = _propagate(t1, src3, dst3, zeros2)
    t2 = _mid(p, deg)
    q = _propagate(t2, src3, dst3, zeros2)
    return _last(q, p, h0, deg)
